# baseline (speedup 1.0000x reference)
import jax
import jax.numpy as jnp
from jax.experimental import pallas as pl
from jax.experimental.pallas import tpu as pltpu

N, D, H, E, K = 2048, 768, 768, 8, 2


def _dense_body(x_ref, wg_ref, w1_ref, b1_ref, w2_ref, b2_ref, y_ref):
    e = pl.program_id(0)
    x = x_ref[...]

    logits = jnp.dot(x, wg_ref[...], preferred_element_type=jnp.float32)
    cols = jax.lax.broadcasted_iota(jnp.int32, logits.shape, 1)
    m1 = jnp.max(logits, axis=1, keepdims=True)
    a1 = jnp.argmax(logits, axis=1).reshape(-1, 1)
    neg = jnp.full_like(logits, -jnp.inf)
    masked = jnp.where(cols == a1, neg, logits)
    m2 = jnp.max(masked, axis=1, keepdims=True)
    a2 = jnp.argmax(masked, axis=1).reshape(-1, 1)
    t = jnp.exp(m2 - m1)
    w1g = 1.0 / (1.0 + t)
    w2g = t / (1.0 + t)
    gate = jnp.where(a1 == e, w1g, jnp.where(a2 == e, w2g, 0.0))

    xb = x.astype(jnp.bfloat16)
    h = jnp.maximum(
        jnp.dot(xb, w1_ref[0].astype(jnp.bfloat16),
                preferred_element_type=jnp.float32) + b1_ref[0], 0.0)
    o = jnp.dot(h.astype(jnp.bfloat16), w2_ref[0].astype(jnp.bfloat16),
                preferred_element_type=jnp.float32) + b2_ref[0]
    contrib = o * gate

    @pl.when(e == 0)
    def _():
        y_ref[...] = contrib

    @pl.when(e != 0)
    def _():
        y_ref[...] = y_ref[...] + contrib


def kernel(x, Wg, W1, b1, W2, b2):
    return pl.pallas_call(
        _dense_body,
        grid=(E,),
        in_specs=[
            pl.BlockSpec((N, D), lambda e: (0, 0)),
            pl.BlockSpec((D, E), lambda e: (0, 0)),
            pl.BlockSpec((1, D, H), lambda e: (e, 0, 0)),
            pl.BlockSpec((1, 1, H), lambda e: (e, 0, 0)),
            pl.BlockSpec((1, H, D), lambda e: (e, 0, 0)),
            pl.BlockSpec((1, 1, D), lambda e: (e, 0, 0)),
        ],
        out_specs=pl.BlockSpec((N, D), lambda e: (0, 0)),
        out_shape=jax.ShapeDtypeStruct((N, D), jnp.float32),
    )(x, Wg, W1, b1.reshape(E, 1, H), W2, b2.reshape(E, 1, D))


# speedup vs baseline: 1.0179x; 1.0179x over previous
import jax
import jax.numpy as jnp
from jax.experimental import pallas as pl
from jax.experimental.pallas import tpu as pltpu

N, D, H, E, K = 2048, 768, 768, 8, 2


def _dense_body(x_ref, wg_ref, w1_ref, b1_ref, w2_ref, b2_ref, y_ref, g_ref):
    e = pl.program_id(0)
    x = x_ref[...]

    @pl.when(e == 0)
    def _():
        logits = jnp.dot(x, wg_ref[...], preferred_element_type=jnp.float32)
        cols = jax.lax.broadcasted_iota(jnp.int32, logits.shape, 1)
        m1 = jnp.max(logits, axis=1, keepdims=True)
        a1 = jnp.argmax(logits, axis=1).reshape(-1, 1)
        neg = jnp.full_like(logits, -jnp.inf)
        masked = jnp.where(cols == a1, neg, logits)
        m2 = jnp.max(masked, axis=1, keepdims=True)
        a2 = jnp.argmax(masked, axis=1).reshape(-1, 1)
        t = jnp.exp(m2 - m1)
        w1g = 1.0 / (1.0 + t)
        w2g = t / (1.0 + t)
        g_ref[...] = jnp.where(cols == a1, w1g, jnp.where(cols == a2, w2g, 0.0))

    cols = jax.lax.broadcasted_iota(jnp.int32, (N, E), 1)
    gate = jnp.sum(jnp.where(cols == e, g_ref[...], 0.0), axis=1, keepdims=True)

    xb = x.astype(jnp.bfloat16)
    h = jnp.maximum(
        jnp.dot(xb, w1_ref[0].astype(jnp.bfloat16),
                preferred_element_type=jnp.float32) + b1_ref[0], 0.0)
    o = jnp.dot(h.astype(jnp.bfloat16), w2_ref[0].astype(jnp.bfloat16),
                preferred_element_type=jnp.float32) + b2_ref[0]
    contrib = o * gate

    @pl.when(e == 0)
    def _():
        y_ref[...] = contrib

    @pl.when(e != 0)
    def _():
        y_ref[...] = y_ref[...] + contrib


def kernel(x, Wg, W1, b1, W2, b2):
    return pl.pallas_call(
        _dense_body,
        grid=(E,),
        in_specs=[
            pl.BlockSpec((N, D), lambda e: (0, 0)),
            pl.BlockSpec((D, E), lambda e: (0, 0)),
            pl.BlockSpec((1, D, H), lambda e: (e, 0, 0)),
            pl.BlockSpec((1, 1, H), lambda e: (e, 0, 0)),
            pl.BlockSpec((1, H, D), lambda e: (e, 0, 0)),
            pl.BlockSpec((1, 1, D), lambda e: (e, 0, 0)),
        ],
        out_specs=pl.BlockSpec((N, D), lambda e: (0, 0)),
        out_shape=jax.ShapeDtypeStruct((N, D), jnp.float32),
        scratch_shapes=[pltpu.VMEM((N, E), jnp.float32)],
    )(x, Wg, W1, b1.reshape(E, 1, H), W2, b2.reshape(E, 1, D))
